# baseline (speedup 1.0000x reference)
import jax
import numpy as np
import jax.numpy as jnp
from jax import lax
from jax.experimental import pallas as pl
from jax.experimental.pallas import tpu as pltpu
from jax.experimental.pallas import tpu_sc as plsc

N_NODES = 10000
NPAD = 10240
E = 320000
D = 128
B = 1024
K = 100
NC = 2
NS = 16
NW = NC * NS
CH = 128
NCHUNK = 80
NPHASE = 2
CPP = NCHUNK // NPHASE
EPP = CPP * CH
EPW = NCHUNK * CH
E_PAD = NW * EPW
ROWS_PT = NPAD // NS
PAIRS = B * K
PPW = PAIRS // NW
PC = 128
NPC = PPW // PC

_mesh = plsc.VectorSubcoreMesh(core_axis_name="c", subcore_axis_name="s",
                               num_cores=NC, num_subcores=NS)


def _i32(x):
    return jnp.asarray(x, jnp.int32)


def _worker_id():
    return _i32(lax.axis_index("s")) * _i32(NC) + _i32(lax.axis_index("c"))


def _make_agg(with_deg):
    out_type = [jax.ShapeDtypeStruct((NC, NPAD, D), jnp.float32)]
    scratch = [
        pltpu.VMEM_SHARED((NPAD, D), jnp.float32),
        pltpu.VMEM((EPP,), jnp.int32),
        pltpu.VMEM((CPP, CH), jnp.int32),
        pltpu.VMEM((CH, D), jnp.float32),
        pltpu.VMEM((CH, D), jnp.float32),
        pltpu.SemaphoreType.DMA,
        pltpu.SemaphoreType.DMA,
    ]
    if with_deg:
        out_type.append(jax.ShapeDtypeStruct((NC, NPAD), jnp.float32))
        scratch += [
            pltpu.VMEM_SHARED((NPAD,), jnp.float32),
            pltpu.VMEM((CH,), jnp.float32),
        ]

    def body(h_hbm, src_hbm, dst4_hbm, z2d_hbm, z1d_hbm, *rest):
        if with_deg:
            (part_hbm, degp_hbm, agg_sh, srcidx, dsts, rows0, rows1,
             sem0, sem1, deg_sh, ones) = rest
        else:
            (part_hbm, agg_sh, srcidx, dsts, rows0, rows1,
             sem0, sem1) = rest
        core = lax.axis_index("c")
        sub = lax.axis_index("s")
        wid = _worker_id()
        e0 = wid * _i32(EPW)
        r0 = _i32(sub) * _i32(ROWS_PT)

        pltpu.sync_copy(z2d_hbm, agg_sh.at[pl.ds(r0, ROWS_PT)])
        if with_deg:
            pltpu.sync_copy(z1d_hbm.at[pl.ds(r0, ROWS_PT)],
                            deg_sh.at[pl.ds(r0, ROWS_PT)])
            for j in range(CH // 16):
                ones[pl.ds(j * 16, 16)] = jnp.ones((16,), jnp.float32)

        def _gather(g, rows, sem):
            pltpu.async_copy(
                h_hbm.at[srcidx.at[pl.ds(g * _i32(CH), CH)]], rows, sem)

        def _gwait(rows, sem):
            pltpu.make_async_copy(h_hbm.at[pl.ds(0, CH)], rows, sem).wait()

        def _scat(g, rows):
            pltpu.sync_copy(rows, agg_sh.at[dsts.at[g]], add=True)
            if with_deg:
                pltpu.sync_copy(ones, deg_sh.at[dsts.at[g]], add=True)

        for p in range(NPHASE):
            pltpu.sync_copy(
                src_hbm.at[pl.ds(e0 + _i32(p * EPP), EPP)], srcidx)
            pltpu.sync_copy(dst4_hbm.at[wid, _i32(p)], dsts)
            if p == 0:
                plsc.subcore_barrier()

            _gather(_i32(0), rows0, sem0)

            @pl.loop(jnp.int32(0), jnp.int32(CPP // 2))
            def chunk_loop(t):
                g = _i32(t) * _i32(2)
                _gather(g + 1, rows1, sem1)
                _gwait(rows0, sem0)
                _scat(g, rows0)

                @pl.when(g + _i32(2) < _i32(CPP))
                def _():
                    _gather(g + 2, rows0, sem0)

                _gwait(rows1, sem1)
                _scat(g + 1, rows1)

        plsc.subcore_barrier()
        pltpu.sync_copy(agg_sh.at[pl.ds(r0, ROWS_PT)],
                        part_hbm.at[core, pl.ds(r0, ROWS_PT)])
        if with_deg:
            pltpu.sync_copy(deg_sh.at[pl.ds(r0, ROWS_PT)],
                            degp_hbm.at[core, pl.ds(r0, ROWS_PT)])

    return pl.kernel(body, out_type=out_type, mesh=_mesh,
                     scratch_types=scratch)


_agg_deg = _make_agg(True)
_agg = _make_agg(False)


def _z(i):
    return jnp.zeros_like(i)


def _tc_layer(parts, deg_t, w, b):
    R = 1024

    def body(p_ref, d_ref, w_ref, b_ref, o_ref):
        agg = p_ref[0] + p_ref[1]
        deg = d_ref[:, 0:1] + d_ref[:, 1:2]
        x = agg / jnp.maximum(deg, 1.0)
        y = lax.dot_general(x, w_ref[...], (((1,), (1,)), ((), ())),
                            preferred_element_type=jnp.float32)
        o_ref[...] = jnp.tanh(y + b_ref[...])

    return pl.pallas_call(
        body,
        grid=(NPAD // R,),
        in_specs=[
            pl.BlockSpec((NC, R, D), lambda i: (_z(i), i, _z(i))),
            pl.BlockSpec((R, NC), lambda i: (i, _z(i))),
            pl.BlockSpec((D, D), lambda i: (_z(i), _z(i))),
            pl.BlockSpec((1, D), lambda i: (_z(i), _z(i))),
        ],
        out_specs=pl.BlockSpec((R, D), lambda i: (i, _z(i))),
        out_shape=jax.ShapeDtypeStruct((NPAD, D), jnp.float32),
    )(parts, deg_t, w, b.reshape(1, D))


SSLOT = 3


def _score_body(h_hbm, ui_hbm, ii_hbm, out_hbm, *rest):
    uall, iall = rest[0], rest[1]
    urs = rest[2:2 + SSLOT]
    irs = rest[2 + SSLOT:2 + 2 * SSLOT]
    tmp, outv = rest[2 + 2 * SSLOT], rest[3 + 2 * SSLOT]
    sus = rest[4 + 2 * SSLOT:4 + 3 * SSLOT]
    sis = rest[4 + 3 * SSLOT:4 + 4 * SSLOT]
    p0 = _worker_id() * _i32(PPW)
    pltpu.sync_copy(ui_hbm.at[pl.ds(p0, PPW)], uall)
    pltpu.sync_copy(ii_hbm.at[pl.ds(p0, PPW)], iall)

    def _issue(c, k):
        off = c * _i32(PC)
        pltpu.async_copy(h_hbm.at[uall.at[pl.ds(off, PC)]], urs[k], sus[k])
        pltpu.async_copy(h_hbm.at[iall.at[pl.ds(off, PC)]], irs[k], sis[k])

    def _wait(k):
        pltpu.make_async_copy(h_hbm.at[pl.ds(0, PC)], urs[k], sus[k]).wait()
        pltpu.make_async_copy(h_hbm.at[pl.ds(0, PC)], irs[k], sis[k]).wait()

    def _compute(c, k):
        base = c * _i32(PC)
        ur, ir = urs[k], irs[k]

        @pl.loop(jnp.int32(0), jnp.int32(PC // 4))
        def pair(q4):
            for kk in range(4):
                p = _i32(q4) * _i32(4) + _i32(kk)
                acc = ur[p, pl.ds(0, 16)] * ir[p, pl.ds(0, 16)]
                for j in range(1, D // 16):
                    acc = acc + ur[p, pl.ds(j * 16, 16)] * ir[p, pl.ds(j * 16, 16)]
                tmp[pl.ds(p * _i32(16), 16)] = acc

        @pl.loop(jnp.int32(0), jnp.int32(PC // 16))
        def grp(gq):
            g = _i32(gq)
            fbase = (g * _i32(16) + lax.iota(jnp.int32, 16)) * _i32(16)
            res = plsc.load_gather(tmp, [fbase])
            for l in range(1, 16):
                res = res + plsc.load_gather(
                    tmp, [fbase + jnp.full((16,), l, jnp.int32)])
            outv[pl.ds(base + g * _i32(16), 16)] = res

    for k in range(SSLOT):
        _issue(_i32(k), k)

    @pl.loop(jnp.int32(0), jnp.int32((NPC - 1) // SSLOT))
    def chunk(t):
        for k in range(SSLOT):
            c = _i32(t) * _i32(SSLOT) + _i32(k)
            _wait(k)
            _compute(c, k)

            @pl.when(c + _i32(SSLOT) < _i32(NPC))
            def _():
                _issue(c + _i32(SSLOT), k)

    _wait(0)
    _compute(_i32(NPC - 1), 0)

    pltpu.sync_copy(outv, out_hbm.at[pl.ds(p0, PPW)])


_score = pl.kernel(
    _score_body,
    out_type=jax.ShapeDtypeStruct((PAIRS,), jnp.float32),
    mesh=_mesh,
    compiler_params=pltpu.CompilerParams(needs_layout_passes=False),
    scratch_types=(
        [pltpu.VMEM((PPW,), jnp.int32)] * 2
        + [pltpu.VMEM((PC, D), jnp.float32)] * (2 * SSLOT)
        + [pltpu.VMEM((PC * 16,), jnp.float32),
           pltpu.VMEM((PPW,), jnp.float32)]
        + [pltpu.SemaphoreType.DMA] * (2 * SSLOT)
    ),
)


def kernel(embeddings, W0, b0, W1, b1, edge_index, user_index, item_index):
    h0 = embeddings.astype(jnp.float32)
    npad_e = E_PAD - E
    pad_src = (jnp.arange(npad_e, dtype=jnp.int32) * 37) % N_NODES
    pad_dst = N_NODES + (jnp.arange(npad_e, dtype=jnp.int32) % (NPAD - N_NODES))
    src = jnp.concatenate([edge_index[0].astype(jnp.int32), pad_src])
    dst3 = jnp.concatenate([edge_index[1].astype(jnp.int32), pad_dst]
                           ).reshape(NW, NPHASE, CPP, CH)
    uidx = user_index.reshape(-1).astype(jnp.int32)
    iidx = item_index.reshape(-1).astype(jnp.int32)
    z2d = jnp.zeros((ROWS_PT, D), jnp.float32)
    z1d = jnp.zeros((NPAD,), jnp.float32)

    parts, degparts = _agg_deg(h0, src, dst3, z2d, z1d)
    deg_t = degparts.T
    h1 = _tc_layer(parts, deg_t, W0, b0)
    (parts2,) = _agg(h1, src, dst3, z2d, z1d)
    h2 = _tc_layer(parts2, deg_t, W1, b1)
    return _score(h2, uidx, iidx).reshape(B, K)
